# speedup vs baseline: 5.0498x; 5.0498x over previous
import jax
import jax.numpy as jnp
from jax.experimental import pallas as pl
from jax.experimental.pallas import tpu as pltpu

_BR = 2048


def _body(a_ref, tm_ref, q_ref, o_ref):
    q = q_ref[...]
    sel = a_ref[...] == 1
    tm0 = tm_ref[0:1, :]
    tm1 = tm_ref[1:2, :]
    o_ref[:, :128] = q * jnp.where(sel, tm1[:, :128], tm0[:, :128])
    o_ref[:, 128:] = q * jnp.where(sel, tm1[:, 128:], tm0[:, 128:])


def kernel(ques_emb, pad_answer, transform_matrix):
    B, H, D = ques_emb.shape
    R = B * H
    q2 = ques_emb.reshape(R, D)
    a2 = pad_answer.astype(jnp.int32).reshape(R, 1)
    tm = transform_matrix.astype(jnp.float32)
    grid = (R // _BR,)
    out = pl.pallas_call(
        _body,
        grid=grid,
        in_specs=[
            pl.BlockSpec((_BR, 1), lambda i: (i, 0)),
            pl.BlockSpec((2, 2 * D), lambda i: (0, 0)),
            pl.BlockSpec((_BR, D), lambda i: (i, 0)),
        ],
        out_specs=pl.BlockSpec((_BR, 2 * D), lambda i: (i, 0)),
        out_shape=jax.ShapeDtypeStruct((R, 2 * D), jnp.float32),
        compiler_params=pltpu.CompilerParams(
            dimension_semantics=("arbitrary",),
        ),
    )(a2, tm, q2)
    return out.reshape(B, H, 2 * D)


# speedup vs baseline: 5.5562x; 1.1003x over previous
import jax
import jax.numpy as jnp
from jax.experimental import pallas as pl
from jax.experimental.pallas import tpu as pltpu

_BR = 8192


def _body(a_ref, tm_ref, q_ref, o_ref):
    q = q_ref[...]
    sel = a_ref[...] == 1
    tm0 = tm_ref[0:1, :]
    tm1 = tm_ref[1:2, :]
    o_ref[:, :128] = q * jnp.where(sel, tm1[:, :128], tm0[:, :128])
    o_ref[:, 128:] = q * jnp.where(sel, tm1[:, 128:], tm0[:, 128:])


def kernel(ques_emb, pad_answer, transform_matrix):
    B, H, D = ques_emb.shape
    R = B * H
    q2 = ques_emb.reshape(R, D)
    a2 = pad_answer.astype(jnp.int32).reshape(R, 1)
    tm = transform_matrix.astype(jnp.float32)
    grid = (R // _BR,)
    out = pl.pallas_call(
        _body,
        grid=grid,
        in_specs=[
            pl.BlockSpec((_BR, 1), lambda i: (i, 0)),
            pl.BlockSpec((2, 2 * D), lambda i: (0, 0)),
            pl.BlockSpec((_BR, D), lambda i: (i, 0)),
        ],
        out_specs=pl.BlockSpec((_BR, 2 * D), lambda i: (i, 0)),
        out_shape=jax.ShapeDtypeStruct((R, 2 * D), jnp.float32),
        compiler_params=pltpu.CompilerParams(
            dimension_semantics=("arbitrary",),
        ),
    )(a2, tm, q2)
    return out.reshape(B, H, 2 * D)
